# baseline (speedup 1.0000x reference)
import jax
import jax.numpy as jnp
from jax.experimental import pallas as pl


def _leaky_pallas(h, slope):
    def body(h_ref, o_ref):
        v = h_ref[...]
        o_ref[...] = jnp.where(v > 0, v, slope * v)

    return pl.pallas_call(
        body, out_shape=jax.ShapeDtypeStruct(h.shape, h.dtype)
    )(h)


def _gatv2_layer(x, edge_attr, src, dst, Wl, bl, Wr, br, We, att, bias, n_nodes):
    H, C = att.shape
    xl = (x @ Wl + bl).reshape(x.shape[0], H, C)
    xr = (x @ Wr + br).reshape(x.shape[0], H, C)
    ea = (edge_attr @ We).reshape(edge_attr.shape[0], H, C)
    e = jax.nn.leaky_relu(xl[src] + xr[dst] + ea, negative_slope=0.2)
    logits = jnp.einsum('ehc,hc->eh', e, att)
    m = jax.ops.segment_max(logits, dst, num_segments=n_nodes)
    m = jnp.where(jnp.isfinite(m), m, 0.0)
    p = jnp.exp(logits - m[dst])
    s = jax.ops.segment_sum(p, dst, num_segments=n_nodes)
    alpha = p / (s[dst] + 1e-16)
    out = jax.ops.segment_sum(xl[src] * alpha[:, :, None], dst, num_segments=n_nodes)
    return out.mean(axis=1) + bias


@jax.jit
def _forward(x, edge_index, edge_attr, params):
    src, dst = edge_index[0], edge_index[1]
    n = x.shape[0]
    h = _gatv2_layer(x, edge_attr, src, dst, *params[0:7], n)
    h = _leaky_pallas(h, 0.01)
    h = _gatv2_layer(h, edge_attr, src, dst, *params[7:14], n)
    h = _leaky_pallas(h, 0.01)
    h = _gatv2_layer(h, edge_attr, src, dst, *params[14:21], n)
    return h


def kernel(x, edge_index, edge_attr, W1l, b1l, W1r, b1r, We1, att1, bias1,
           W2l, b2l, W2r, b2r, We2, att2, bias2, W3l, b3l, W3r, b3r, We3,
           att3, bias3):
    params = (W1l, b1l, W1r, b1r, We1, att1, bias1,
              W2l, b2l, W2r, b2r, We2, att2, bias2,
              W3l, b3l, W3r, b3r, We3, att3, bias3)
    return _forward(x, edge_index, edge_attr, params)


# speedup vs baseline: 2.3607x; 2.3607x over previous
import functools

import jax
import jax.numpy as jnp
from jax import lax
from jax.experimental import pallas as pl
from jax.experimental.pallas import tpu as pltpu
from jax.experimental.pallas import tpu_sc as plsc

N_NODES = 10000
N_EDGES = 160000
HEADS = 4

_NS = 16
_NC = 2
_B = 256
_W = 128
_EPT = 10240
_EPAD = _EPT * _NS
_NB = _EPT // _B
_NP = 10240
_RPT = _NP // _NS


@functools.partial(jax.jit, static_argnames=("num_chunks",))
def _sc_weighted_scatter(table, walpha, src, dst, num_chunks):
    K = num_chunks
    K2 = K // _NC
    zeros = jnp.zeros((_RPT, _W), jnp.float32)

    mesh = plsc.VectorSubcoreMesh(core_axis_name="c", subcore_axis_name="s")

    @functools.partial(
        pl.kernel,
        out_type=jax.ShapeDtypeStruct((K, _NP, _W), jnp.float32),
        mesh=mesh,
        scratch_types=[
            pltpu.VMEM_SHARED((_NP, _W), jnp.float32),
            pltpu.VMEM((_B,), jnp.int32),
            pltpu.VMEM((_B,), jnp.int32),
            pltpu.VMEM((_B,), jnp.float32),
            pltpu.VMEM((_B, _W), jnp.float32),
            pltpu.SemaphoreType.DMA,
        ],
    )
    def scatter_kernel(table_hbm, walpha_hbm, src_hbm, dst_hbm, zeros_hbm,
                       out_hbm, acc, idx_v, didx_v, w_v, rows_v, sem):
        c = lax.axis_index("c")
        s = lax.axis_index("s")

        @pl.loop(0, K2)
        def chunk_loop(kl):
            k = c * K2 + kl
            pltpu.sync_copy(zeros_hbm, acc.at[pl.ds(s * _RPT, _RPT)])
            plsc.subcore_barrier()

            @pl.loop(0, _NB)
            def batch_loop(j):
                e0 = s * _EPT + j * _B
                pltpu.sync_copy(src_hbm.at[pl.ds(e0, _B)], idx_v)
                pltpu.sync_copy(dst_hbm.at[pl.ds(e0, _B)], didx_v)
                pltpu.sync_copy(walpha_hbm.at[pl.ds(k * _EPAD + e0, _B)], w_v)
                base = k * N_NODES

                @pl.loop(0, _B // 16)
                def rebase_loop(t):
                    sl = pl.ds(t * 16, 16)
                    idx_v[sl] = idx_v[sl] + base

                pltpu.async_copy(table_hbm.at[idx_v], rows_v, sem).wait()

                @pl.loop(0, _B // 16)
                def mul_loop(g):
                    wvec = w_v[pl.ds(g * 16, 16)]
                    for l in range(16):
                        wv = jnp.full((16,), wvec[l], jnp.float32)
                        i = g * 16 + l
                        for r in range(_W // 16):
                            sl = pl.ds(r * 16, 16)
                            rows_v[i, sl] = rows_v[i, sl] * wv

                pltpu.sync_copy(rows_v, acc.at[didx_v], add=True)

            plsc.subcore_barrier()
            pltpu.sync_copy(acc.at[pl.ds(s * _RPT, _RPT)],
                            out_hbm.at[k, pl.ds(s * _RPT, _RPT)])
            plsc.subcore_barrier()

    return scatter_kernel(table, walpha, src, dst, zeros)


def _gatv2_layer(x, edge_attr, src, dst, Wl, bl, Wr, br, We, att, bias, n_nodes):
    H, C = att.shape
    CK = -(-C // _W)
    Cp = CK * _W
    K = H * CK
    xl = x @ Wl + bl
    xr = x @ Wr + br
    ea = edge_attr @ We
    xl3 = xl.reshape(n_nodes, H, C)
    e = jax.nn.leaky_relu(xl3[src] + xr.reshape(n_nodes, H, C)[dst]
                          + ea.reshape(-1, H, C), negative_slope=0.2)
    logits = jnp.einsum('ehc,hc->eh', e, att)
    m = jax.ops.segment_max(logits, dst, num_segments=n_nodes)
    m = jnp.where(jnp.isfinite(m), m, 0.0)
    p = jnp.exp(logits - m[dst])
    ssum = jax.ops.segment_sum(p, dst, num_segments=n_nodes)
    alpha = p / (ssum[dst] + 1e-16)

    xlp = jnp.pad(xl3, ((0, 0), (0, 0), (0, Cp - C)))
    table = (xlp.reshape(n_nodes, H, CK, _W)
             .transpose(1, 2, 0, 3)
             .reshape(K * n_nodes, _W))
    alphat = jnp.pad(alpha.T, ((0, 0), (0, _EPAD - alpha.shape[0])))
    walpha = jnp.repeat(alphat, CK, axis=0).reshape(-1)
    srcp = jnp.pad(src, (0, _EPAD - src.shape[0]))
    dstp = jnp.pad(dst, (0, _EPAD - dst.shape[0]))
    out_chunks = _sc_weighted_scatter(table, walpha, srcp, dstp, num_chunks=K)
    out = (out_chunks[:, :n_nodes].reshape(H, CK, n_nodes, _W)
           .transpose(2, 0, 1, 3)
           .reshape(n_nodes, H, Cp)[:, :, :C])
    return out.mean(axis=1) + bias


@jax.jit
def _forward(x, edge_index, edge_attr, params):
    src, dst = edge_index[0], edge_index[1]
    n = x.shape[0]
    h = _gatv2_layer(x, edge_attr, src, dst, *params[0:7], n)
    h = jax.nn.leaky_relu(h, negative_slope=0.01)
    h = _gatv2_layer(h, edge_attr, src, dst, *params[7:14], n)
    h = jax.nn.leaky_relu(h, negative_slope=0.01)
    h = _gatv2_layer(h, edge_attr, src, dst, *params[14:21], n)
    return h


def kernel(x, edge_index, edge_attr, W1l, b1l, W1r, b1r, We1, att1, bias1,
           W2l, b2l, W2r, b2r, We2, att2, bias2, W3l, b3l, W3r, b3r, We3,
           att3, bias3):
    params = (W1l, b1l, W1r, b1r, We1, att1, bias1,
              W2l, b2l, W2r, b2r, We2, att2, bias2,
              W3l, b3l, W3r, b3r, We3, att3, bias3)
    return _forward(x, edge_index, edge_attr, params)
